# speedup vs baseline: 4.5756x; 4.5756x over previous
import functools
import math

import jax
import jax.numpy as jnp
from jax import lax
from jax.experimental import pallas as pl
from jax.experimental.pallas import tpu as pltpu
from jax.experimental.pallas import tpu_sc as plsc

D_MODEL = 768
_SCALE = math.sqrt(D_MODEL)

_info = plsc.get_sparse_core_info()
_NC = _info.num_cores
_NS = _info.num_subcores
_L = _info.num_lanes
_NW = _NC * _NS

_CHUNK = 64


def _make_kernel(B: int):
    assert B % (_NW * _CHUNK) == 0
    b_per_w = B // _NW
    n_chunks = b_per_w // _CHUNK
    n_vecs = D_MODEL // _L

    mesh = plsc.VectorSubcoreMesh(core_axis_name="c", subcore_axis_name="s")

    @functools.partial(
        pl.kernel,
        mesh=mesh,
        out_type=jax.ShapeDtypeStruct((B, D_MODEL), jnp.float32),
        scratch_types=[
            pltpu.VMEM((n_chunks, _CHUNK), jnp.int32),
            pltpu.VMEM((_CHUNK, D_MODEL), jnp.float32),
            pltpu.VMEM((_CHUNK, D_MODEL), jnp.float32),
            pltpu.SemaphoreType.DMA,
            pltpu.SemaphoreType.DMA,
            pltpu.SemaphoreType.DMA,
            pltpu.SemaphoreType.DMA,
        ],
    )
    def emb_kernel(table_hbm, x_hbm, out_hbm, idx_v, buf0, buf1,
                   gsem0, gsem1, osem0, osem1):
        wid = lax.axis_index("s") * _NC + lax.axis_index("c")
        base = wid * b_per_w

        pltpu.sync_copy(x_hbm.at[wid], idx_v)

        bufs = (buf0, buf1)
        gsems = (gsem0, gsem1)
        osems = (osem0, osem1)

        def scale_chunk(buf):
            def row_body(r, carry):
                for j in range(n_vecs):
                    sl = (r, pl.ds(j * _L, _L))
                    buf[sl] = buf[sl] * _SCALE
                return carry
            lax.fori_loop(0, _CHUNK, row_body, 0)

        gcp = [None, None]
        ocp = [None, None]
        gcp[0] = pltpu.async_copy(table_hbm.at[idx_v.at[0]], buf0, gsem0)
        for g in range(n_chunks):
            b = g % 2
            nb = (g + 1) % 2
            gcp[b].wait()
            if g + 1 < n_chunks:
                if ocp[nb] is not None:
                    ocp[nb].wait()
                gcp[nb] = pltpu.async_copy(
                    table_hbm.at[idx_v.at[g + 1]], bufs[nb], gsems[nb])
            scale_chunk(bufs[b])
            ocp[b] = pltpu.async_copy(
                bufs[b], out_hbm.at[pl.ds(base + g * _CHUNK, _CHUNK)],
                osems[b])
        ocp[0].wait()
        ocp[1].wait()

    return emb_kernel


def kernel(table, x):
    B = x.size
    x_blocked = x.reshape(_NW, B // _NW // _CHUNK, _CHUNK)
    out = _make_kernel(B)(table, x_blocked)
    return out.reshape(x.shape + (D_MODEL,))


# speedup vs baseline: 4.7932x; 1.0475x over previous
import functools
import math

import jax
import jax.numpy as jnp
from jax import lax
from jax.experimental import pallas as pl
from jax.experimental.pallas import tpu as pltpu
from jax.experimental.pallas import tpu_sc as plsc

D_MODEL = 768
_SCALE = math.sqrt(D_MODEL)

_info = plsc.get_sparse_core_info()
_NC = _info.num_cores
_NS = _info.num_subcores
_L = _info.num_lanes
_NW = _NC * _NS

_CHUNK = 32


def _make_kernel(B: int):
    assert B % (_NW * _CHUNK) == 0
    b_per_w = B // _NW
    n_chunks = b_per_w // _CHUNK
    assert n_chunks >= 4 and n_chunks % 2 == 0
    n_vecs = D_MODEL // _L

    mesh = plsc.VectorSubcoreMesh(core_axis_name="c", subcore_axis_name="s")

    @functools.partial(
        pl.kernel,
        mesh=mesh,
        out_type=jax.ShapeDtypeStruct((B, D_MODEL), jnp.float32),
        scratch_types=[
            pltpu.VMEM((n_chunks, _CHUNK), jnp.int32),
            pltpu.VMEM((_CHUNK, D_MODEL), jnp.float32),
            pltpu.VMEM((_CHUNK, D_MODEL), jnp.float32),
            pltpu.VMEM((_CHUNK, D_MODEL), jnp.float32),
            pltpu.VMEM((_CHUNK, D_MODEL), jnp.float32),
            pltpu.SemaphoreType.DMA,
            pltpu.SemaphoreType.DMA,
            pltpu.SemaphoreType.DMA,
            pltpu.SemaphoreType.DMA,
        ],
    )
    def emb_kernel(table_hbm, x_hbm, out_hbm, idx_v, gbuf0, gbuf1,
                   obuf0, obuf1, gsem0, gsem1, osem0, osem1):
        wid = lax.axis_index("s") * _NC + lax.axis_index("c")
        base = wid * b_per_w

        gbufs = (gbuf0, gbuf1)
        obufs = (obuf0, obuf1)
        gsems = (gsem0, gsem1)
        osems = (osem0, osem1)

        pltpu.sync_copy(x_hbm.at[wid], idx_v)

        def issue_gather(c, b):
            return pltpu.async_copy(
                table_hbm.at[idx_v.at[c]], gbufs[b], gsems[b])

        def wait_gather(b):
            pltpu.make_async_copy(
                table_hbm.at[idx_v.at[0]], gbufs[b], gsems[b]).wait()

        def issue_write(c, b):
            return pltpu.async_copy(
                obufs[b], out_hbm.at[pl.ds(base + c * _CHUNK, _CHUNK)],
                osems[b])

        def wait_write(b):
            pltpu.make_async_copy(
                obufs[b], out_hbm.at[pl.ds(base, _CHUNK)], osems[b]).wait()

        def scale(b):
            src = gbufs[b]
            dst = obufs[b]
            def row_body(r, carry):
                for j in range(n_vecs):
                    sl = (r, pl.ds(j * _L, _L))
                    dst[sl] = src[sl] * _SCALE
                return carry
            lax.fori_loop(0, _CHUNK, row_body, 0)

        issue_gather(0, 0)
        issue_gather(1, 1)
        for b in (0, 1):
            wait_gather(b)
            scale(b)
            issue_gather(b + 2, b)
            issue_write(b, b)

        def loop_body(i, carry):
            g = 2 + 2 * i
            for b in (0, 1):
                c = g + b
                wait_gather(b)
                wait_write(b)
                scale(b)
                issue_gather(c + 2, b)
                issue_write(c, b)
            return carry
        lax.fori_loop(0, (n_chunks - 4) // 2, loop_body, 0)

        for b in (0, 1):
            c = n_chunks - 2 + b
            wait_gather(b)
            wait_write(b)
            scale(b)
            issue_write(c, b)
        wait_write(0)
        wait_write(1)

    return emb_kernel


def kernel(table, x):
    B = x.size
    x_blocked = x.reshape(_NW, B // _NW // _CHUNK, _CHUNK)
    out = _make_kernel(B)(table, x_blocked)
    return out.reshape(x.shape + (D_MODEL,))
